# Initial kernel scaffold; baseline (speedup 1.0000x reference)
import jax
import jax.numpy as jnp
from jax.experimental import pallas as pl


def kernel(node_features, edge_index, relation_types, W0, a_src0, a_dst0, W1, a_src1, a_dst1):
    raise NotImplementedError("write your pallas kernel here")


# speedup vs baseline: 2.9800x; 2.9800x over previous
import functools

import jax
import jax.numpy as jnp
from jax.experimental import pallas as pl


def _scores_kernel(h_ref, b_ref, o_ref):
    o_ref[:] = jnp.dot(h_ref[:], b_ref[:], preferred_element_type=jnp.float32)


def _combine_kernel(s_ref, w_ref, o_ref, *, nrel, apply_elu):
    def body(r, acc):
        return acc + jnp.dot(s_ref[r], w_ref[r],
                             preferred_element_type=jnp.float32)

    acc = jax.lax.fori_loop(
        0, nrel, body, jnp.zeros(o_ref.shape, jnp.float32))
    if apply_elu:
        acc = jnp.where(acc > 0, acc, jnp.exp(jnp.minimum(acc, 0.0)) - 1.0)
    o_ref[:] = acc


def _node_scores(h, b_padded, bn):
    n, d_in = h.shape
    return pl.pallas_call(
        _scores_kernel,
        grid=(n // bn,),
        in_specs=[
            pl.BlockSpec((bn, d_in), lambda i: (i, 0)),
            pl.BlockSpec((d_in, 128), lambda i: (0, 0)),
        ],
        out_specs=pl.BlockSpec((bn, 128), lambda i: (i, 0)),
        out_shape=jax.ShapeDtypeStruct((n, 128), jnp.float32),
    )(h, b_padded)


def _combine(s, w, bn, apply_elu):
    nrel, n, d_in = s.shape
    d_out = w.shape[-1]
    kern = functools.partial(_combine_kernel, nrel=nrel, apply_elu=apply_elu)
    return pl.pallas_call(
        kern,
        grid=(n // bn,),
        in_specs=[
            pl.BlockSpec((nrel, bn, d_in), lambda i: (0, i, 0)),
            pl.BlockSpec((nrel, d_in, d_out), lambda i: (0, 0, 0)),
        ],
        out_specs=pl.BlockSpec((bn, d_out), lambda i: (i, 0)),
        out_shape=jax.ShapeDtypeStruct((n, d_out), jnp.float32),
    )(s, w)


def _rgat_layer(h, src, dst, rel, W, a_s, a_d, apply_elu):
    n, d_in = h.shape
    nrel = W.shape[0]
    bn = 1000

    b_s = jnp.einsum('rde,re->rd', W, a_s)
    b_d = jnp.einsum('rde,re->rd', W, a_d)
    b = jnp.concatenate([b_s, b_d], axis=0).T
    b_padded = jnp.zeros((d_in, 128), jnp.float32).at[:, :2 * nrel].set(b)

    scores = _node_scores(h, b_padded, bn)

    e_src = scores[src, rel]
    e_dst = scores[dst, nrel + rel]
    e = e_src + e_dst
    e = jnp.where(e > 0, e, 0.2 * e)

    m = jax.ops.segment_max(e, dst, num_segments=n)
    m = jnp.where(jnp.isfinite(m), m, 0.0)
    ex = jnp.exp(e - m[dst])
    denom = jax.ops.segment_sum(ex, dst, num_segments=n)
    alpha = ex / (denom[dst] + 1e-9)

    seg = rel * n + dst
    s = jax.ops.segment_sum(alpha[:, None] * h[src], seg,
                            num_segments=nrel * n)
    s = s.reshape(nrel, n, d_in)

    return _combine(s, W, bn, apply_elu)


def kernel(node_features, edge_index, relation_types,
           W0, a_src0, a_dst0, W1, a_src1, a_dst1):
    src = edge_index[0]
    dst = edge_index[1]
    rel = relation_types
    h = _rgat_layer(node_features, src, dst, rel, W0, a_src0, a_dst0,
                    apply_elu=True)
    h = _rgat_layer(h, src, dst, rel, W1, a_src1, a_dst1,
                    apply_elu=False)
    return h


# speedup vs baseline: 2.9869x; 1.0023x over previous
import functools

import jax
import jax.numpy as jnp
from jax.experimental import pallas as pl


def _scores_kernel(h_ref, b_ref, o_ref):
    o_ref[:] = jnp.dot(h_ref[:], b_ref[:], preferred_element_type=jnp.float32)


def _combine_kernel(s_ref, w_ref, o_ref, *, nrel, apply_elu):
    def body(r, acc):
        return acc + jnp.dot(s_ref[r], w_ref[r],
                             preferred_element_type=jnp.float32)

    acc = jax.lax.fori_loop(
        0, nrel, body, jnp.zeros(o_ref.shape, jnp.float32))
    if apply_elu:
        acc = jnp.where(acc > 0, acc, jnp.exp(jnp.minimum(acc, 0.0)) - 1.0)
    o_ref[:] = acc


def _node_scores(h, b_padded, bn):
    n, d_in = h.shape
    return pl.pallas_call(
        _scores_kernel,
        grid=(n // bn,),
        in_specs=[
            pl.BlockSpec((bn, d_in), lambda i: (i, 0)),
            pl.BlockSpec((d_in, 128), lambda i: (0, 0)),
        ],
        out_specs=pl.BlockSpec((bn, 128), lambda i: (i, 0)),
        out_shape=jax.ShapeDtypeStruct((n, 128), jnp.float32),
    )(h, b_padded)


def _combine(s, w, bn, apply_elu):
    nrel, n, d_in = s.shape
    d_out = w.shape[-1]
    kern = functools.partial(_combine_kernel, nrel=nrel, apply_elu=apply_elu)
    return pl.pallas_call(
        kern,
        grid=(n // bn,),
        in_specs=[
            pl.BlockSpec((nrel, bn, d_in), lambda i: (0, i, 0)),
            pl.BlockSpec((nrel, d_in, d_out), lambda i: (0, 0, 0)),
        ],
        out_specs=pl.BlockSpec((bn, d_out), lambda i: (i, 0)),
        out_shape=jax.ShapeDtypeStruct((n, d_out), jnp.float32),
    )(s, w)


def _rgat_layer(h, src, dst, rel, W, a_s, a_d, apply_elu):
    n, d_in = h.shape
    nrel = W.shape[0]
    bn = 1000

    b_s = jnp.einsum('rde,re->rd', W, a_s)
    b_d = jnp.einsum('rde,re->rd', W, a_d)
    b = jnp.concatenate([b_s, b_d], axis=0).T
    b_padded = jnp.zeros((d_in, 128), jnp.float32).at[:, :2 * nrel].set(b)

    scores = _node_scores(h, b_padded, bn)

    e_src = scores[src, rel]
    e_dst = scores[dst, nrel + rel]
    e = e_src + e_dst
    e = jnp.where(e > 0, e, 0.2 * e)

    m = jax.ops.segment_max(e, dst, num_segments=n)
    m = jnp.where(jnp.isfinite(m), m, 0.0)
    ex = jnp.exp(e - m[dst])
    denom = jax.ops.segment_sum(ex, dst, num_segments=n)
    alpha = ex / (denom[dst] + 1e-9)

    seg = rel * n + dst
    h_gathered = h.astype(jnp.bfloat16)[src].astype(jnp.float32)
    s = jax.ops.segment_sum(alpha[:, None] * h_gathered, seg,
                            num_segments=nrel * n)
    s = s.reshape(nrel, n, d_in)

    return _combine(s, W, bn, apply_elu)


def kernel(node_features, edge_index, relation_types,
           W0, a_src0, a_dst0, W1, a_src1, a_dst1):
    src = edge_index[0]
    dst = edge_index[1]
    rel = relation_types
    h = _rgat_layer(node_features, src, dst, rel, W0, a_src0, a_dst0,
                    apply_elu=True)
    h = _rgat_layer(h, src, dst, rel, W1, a_src1, a_dst1,
                    apply_elu=False)
    return h
